# baseline (speedup 1.0000x reference)
import jax
import jax.numpy as jnp
from jax import lax
from jax.experimental import pallas as pl
from jax.experimental.pallas import tpu as pltpu
from jax.experimental.pallas import tpu_sc as plsc

SEQ_LEN = 200
EMBED_DIM = 64
CHUNK = 100
NC = 2
NS = 16
NW = NC * NS


def _body(x_hbm, pos_hbm, tok_hbm, out_hbm, idx_v, pos_v, rows_v, gsem):
    nchunks = x_hbm.shape[0] // NW
    wid = lax.axis_index("s") * NC + lax.axis_index("c")

    pltpu.sync_copy(x_hbm.at[pl.ds(wid * nchunks, nchunks)], idx_v)
    pltpu.sync_copy(pos_hbm, pos_v)

    def chunk_step(c, _):
        pltpu.async_copy(tok_hbm.at[idx_v.at[c]], rows_v, gsem).wait()

        pbase = (c % 2) * CHUNK

        def row_add(r, carry):
            for j in range(EMBED_DIM // 16):
                sl = pl.ds(16 * j, 16)
                rows_v[r, sl] += pos_v[pbase + r, sl]
            return carry

        lax.fori_loop(0, CHUNK, row_add, 0)

        row_lo = (wid * nchunks + c) * CHUNK
        pltpu.sync_copy(rows_v, out_hbm.at[pl.ds(row_lo, CHUNK)])
        return 0

    lax.fori_loop(0, nchunks, chunk_step, 0)


def _make_kernel(n_flat, nchunks_per_worker):
    mesh = plsc.VectorSubcoreMesh(core_axis_name="c", subcore_axis_name="s")
    return pl.kernel(
        _body,
        out_type=jax.ShapeDtypeStruct((n_flat, EMBED_DIM), jnp.float32),
        mesh=mesh,
        scratch_types=[
            pltpu.VMEM((nchunks_per_worker, CHUNK), jnp.int32),
            pltpu.VMEM((SEQ_LEN, EMBED_DIM), jnp.float32),
            pltpu.VMEM((CHUNK, EMBED_DIM), jnp.float32),
            pltpu.SemaphoreType.DMA,
        ],
        compiler_params=pltpu.CompilerParams(use_tc_tiling_on_sc=False),
    )


def kernel(x, token_table, pos_table):
    batch, seq = x.shape
    assert seq == SEQ_LEN
    n_flat = batch * seq
    nchunks = n_flat // (NW * CHUNK)
    x_chunks = x.reshape(n_flat // CHUNK, CHUNK).astype(jnp.int32)
    run = _make_kernel(n_flat, nchunks)
    out = run(x_chunks, pos_table, token_table)
    return out.reshape(batch, seq, EMBED_DIM)


# speedup vs baseline: 1.2045x; 1.2045x over previous
import jax
import jax.numpy as jnp
from jax import lax
from jax.experimental import pallas as pl
from jax.experimental.pallas import tpu as pltpu
from jax.experimental.pallas import tpu_sc as plsc

SEQ_LEN = 200
EMBED_DIM = 64
CHUNK = 100
NC = 2
NS = 16
NW = NC * NS


def _body(x_hbm, pos_hbm, tok_hbm, out_hbm,
          idx_v, pos_v, in0, in1, out0, out1, g0, g1, s0, s1):
    ins, outs, gsems, ssems = [in0, in1], [out0, out1], [g0, g1], [s0, s1]
    nchunks = x_hbm.shape[0] // NW
    wid = lax.axis_index("s") * NC + lax.axis_index("c")
    base_chunk = wid * nchunks

    pltpu.sync_copy(x_hbm.at[pl.ds(base_chunk, nchunks)], idx_v)
    pltpu.sync_copy(pos_hbm, pos_v)

    for b in range(2):
        pltpu.async_copy(tok_hbm.at[idx_v.at[b]], ins[b], gsems[b])

    def outer(t, carry):
        for b in range(2):
            cc = 2 * t + b
            pltpu.make_async_copy(tok_hbm.at[idx_v.at[cc]], ins[b],
                                  gsems[b]).wait()

            @pl.when(t >= 1)
            def _wait_store():
                pltpu.make_async_copy(outs[b], out_hbm.at[pl.ds(0, CHUNK)],
                                      ssems[b]).wait()

            pbase = b * CHUNK

            def row_add(r, c2):
                for j in range(EMBED_DIM // 16):
                    sl = pl.ds(16 * j, 16)
                    outs[b][r, sl] = ins[b][r, sl] + pos_v[pbase + r, sl]
                return c2

            lax.fori_loop(0, CHUNK, row_add, 0, unroll=4)

            row_lo = (base_chunk + cc) * CHUNK
            pltpu.async_copy(outs[b], out_hbm.at[pl.ds(row_lo, CHUNK)],
                             ssems[b])

            @pl.when(cc + 2 < nchunks)
            def _next_gather():
                pltpu.async_copy(tok_hbm.at[idx_v.at[cc + 2]], ins[b],
                                 gsems[b])
        return carry

    lax.fori_loop(0, nchunks // 2, outer, 0)

    for b in range(2):
        pltpu.make_async_copy(outs[b], out_hbm.at[pl.ds(0, CHUNK)],
                              ssems[b]).wait()


def _make_kernel(n_flat, nchunks_per_worker):
    mesh = plsc.VectorSubcoreMesh(core_axis_name="c", subcore_axis_name="s")
    return pl.kernel(
        _body,
        out_type=jax.ShapeDtypeStruct((n_flat, EMBED_DIM), jnp.float32),
        mesh=mesh,
        scratch_types=[
            pltpu.VMEM((nchunks_per_worker, CHUNK), jnp.int32),
            pltpu.VMEM((SEQ_LEN, EMBED_DIM), jnp.float32),
            pltpu.VMEM((CHUNK, EMBED_DIM), jnp.float32),
            pltpu.VMEM((CHUNK, EMBED_DIM), jnp.float32),
            pltpu.VMEM((CHUNK, EMBED_DIM), jnp.float32),
            pltpu.VMEM((CHUNK, EMBED_DIM), jnp.float32),
            pltpu.SemaphoreType.DMA,
            pltpu.SemaphoreType.DMA,
            pltpu.SemaphoreType.DMA,
            pltpu.SemaphoreType.DMA,
        ],
        compiler_params=pltpu.CompilerParams(use_tc_tiling_on_sc=False),
    )


def kernel(x, token_table, pos_table):
    batch, seq = x.shape
    assert seq == SEQ_LEN
    n_flat = batch * seq
    nchunks = n_flat // (NW * CHUNK)
    x_chunks = x.reshape(n_flat // CHUNK, CHUNK).astype(jnp.int32)
    run = _make_kernel(n_flat, nchunks)
    out = run(x_chunks, pos_table, token_table)
    return out.reshape(batch, seq, EMBED_DIM)


# speedup vs baseline: 1.2071x; 1.0021x over previous
import jax
import jax.numpy as jnp
from jax import lax
from jax.experimental import pallas as pl
from jax.experimental.pallas import tpu as pltpu
from jax.experimental.pallas import tpu_sc as plsc

SEQ_LEN = 200
EMBED_DIM = 64
CHUNK = 100
NC = 2
NS = 16
NW = NC * NS


def _body(x_hbm, pos_hbm, tok_hbm, out_hbm,
          idx_v, pos_v, in0, in1, out0, out1, g0, g1, s0, s1):
    ins, outs, gsems, ssems = [in0, in1], [out0, out1], [g0, g1], [s0, s1]
    rows = x_hbm.shape[0] // NW
    wid = lax.axis_index("s") * NC + lax.axis_index("c")
    row_base = wid * rows

    pltpu.sync_copy(x_hbm.at[pl.ds(row_base, rows)], idx_v)
    pltpu.sync_copy(pos_hbm, pos_v)

    SPLITS = ((0, 128), (128, SEQ_LEN - 128))

    def gather_row(t, p):
        for lo, n in SPLITS:
            pltpu.async_copy(
                tok_hbm.at[idx_v.at[t, pl.ds(lo, n)]],
                ins[p].at[pl.ds(lo, n)], gsems[p])

    def wait_row(t, p):
        for lo, n in SPLITS:
            pltpu.make_async_copy(
                tok_hbm.at[idx_v.at[t, pl.ds(lo, n)]],
                ins[p].at[pl.ds(lo, n)], gsems[p]).wait()

    for p in range(2):
        gather_row(p, p)

    def outer(u, carry):
        for p in range(2):
            t = 2 * u + p
            wait_row(t, p)

            @pl.when(u >= 1)
            def _wait_store():
                pltpu.make_async_copy(outs[p], out_hbm.at[0], ssems[p]).wait()

            def row_add(r, c2):
                for j in range(EMBED_DIM // 16):
                    sl = pl.ds(16 * j, 16)
                    outs[p][r, sl] = ins[p][r, sl] + pos_v[r, sl]
                return c2

            lax.fori_loop(0, SEQ_LEN, row_add, 0, unroll=4)

            pltpu.async_copy(outs[p], out_hbm.at[row_base + t], ssems[p])

            @pl.when(t + 2 < rows)
            def _next_gather():
                gather_row(t + 2, p)
        return carry

    lax.fori_loop(0, rows // 2, outer, 0)

    for p in range(2):
        pltpu.make_async_copy(outs[p], out_hbm.at[0], ssems[p]).wait()


def _make_kernel(batch):
    mesh = plsc.VectorSubcoreMesh(core_axis_name="c", subcore_axis_name="s")
    rows = batch // NW
    return pl.kernel(
        _body,
        out_type=jax.ShapeDtypeStruct((batch, SEQ_LEN, EMBED_DIM), jnp.float32),
        mesh=mesh,
        scratch_types=[
            pltpu.VMEM((rows, SEQ_LEN), jnp.int32),
            pltpu.VMEM((SEQ_LEN, EMBED_DIM), jnp.float32),
            pltpu.VMEM((SEQ_LEN, EMBED_DIM), jnp.float32),
            pltpu.VMEM((SEQ_LEN, EMBED_DIM), jnp.float32),
            pltpu.VMEM((SEQ_LEN, EMBED_DIM), jnp.float32),
            pltpu.VMEM((SEQ_LEN, EMBED_DIM), jnp.float32),
            pltpu.SemaphoreType.DMA,
            pltpu.SemaphoreType.DMA,
            pltpu.SemaphoreType.DMA,
            pltpu.SemaphoreType.DMA,
        ],
        compiler_params=pltpu.CompilerParams(use_tc_tiling_on_sc=False),
    )


def kernel(x, token_table, pos_table):
    batch, seq = x.shape
    assert seq == SEQ_LEN
    run = _make_kernel(batch)
    return run(x.astype(jnp.int32), pos_table, token_table)


# speedup vs baseline: 1.3434x; 1.1129x over previous
import jax
import jax.numpy as jnp
from jax import lax
from jax.experimental import pallas as pl
from jax.experimental.pallas import tpu as pltpu
from jax.experimental.pallas import tpu_sc as plsc

SEQ_LEN = 200
EMBED_DIM = 64
NC = 2
NS = 16
NW = NC * NS

H0, H1 = 104, SEQ_LEN - 104
WIN0 = tuple(range(0, 96, 16)) + (88,)
WIN1 = tuple(range(H0, SEQ_LEN - 16 + 1, 16))


def _body(x_hbm, pos_hbm, tok_hbm, out_hbm,
          idx_v, pos_v, gb0, gb1, in0, in1, out0, out1, g0, g1, s0, s1):
    gbs, ins, outs = [gb0, gb1], [in0, in1], [out0, out1]
    gsems, ssems = [g0, g1], [s0, s1]
    los, lens = (0, H0), (H0, H1)
    rows = x_hbm.shape[0] // NW
    wid = lax.axis_index("s") * NC + lax.axis_index("c")
    row_base = wid * rows

    pltpu.sync_copy(x_hbm.at[pl.ds(row_base, rows)], idx_v)
    pltpu.sync_copy(pos_hbm, pos_v)

    def fill_gidx(t, p):
        for o in (WIN0, WIN1)[p]:
            gbs[p][pl.ds(o - los[p], 16)] = lax.shift_right_logical(
                idx_v[t, pl.ds(o, 16)], 1)

    def gather_half(p):
        pltpu.async_copy(tok_hbm.at[gbs[p].at[pl.ds(0, lens[p])]],
                         ins[p], gsems[p])

    def wait_half(p):
        pltpu.make_async_copy(tok_hbm.at[gbs[p].at[pl.ds(0, lens[p])]],
                              ins[p], gsems[p]).wait()

    def add_window(t, p, o, lanes):
        toks = idx_v[t, pl.ds(o, 16)]
        for l in lanes:
            r = o + l
            rb = r - los[p]
            off = (toks[l] & 1) * EMBED_DIM
            for j in range(EMBED_DIM // 16):
                sl = pl.ds(16 * j, 16)
                src = pl.multiple_of(off + 16 * j, 16)
                outs[p][rb, sl] = ins[p][rb, pl.ds(src, 16)] + pos_v[r, sl]

    for p in range(2):
        fill_gidx(0, p)
        gather_half(p)

    def outer(u, carry):
        for p in range(2):
            wait_half(p)

            @pl.when(u >= 1)
            def _wait_store():
                pltpu.make_async_copy(
                    outs[p], out_hbm.at[0, pl.ds(0, lens[p])],
                    ssems[p]).wait()

            if p == 0:
                @plsc.parallel_loop(0, len(WIN0) - 1)
                def _half0(g):
                    add_window(u, 0, pl.multiple_of(g * 16, 16), range(16))
                add_window(u, 0, WIN0[-1], range(8, 16))
            else:
                for o in WIN1:
                    add_window(u, 1, o, range(16))

            pltpu.async_copy(
                outs[p], out_hbm.at[row_base + u, pl.ds(los[p], lens[p])],
                ssems[p])

            @pl.when(u + 1 < rows)
            def _next_gather():
                fill_gidx(u + 1, p)
                gather_half(p)
        return carry

    lax.fori_loop(0, rows, outer, 0)

    for p in range(2):
        pltpu.make_async_copy(outs[p], out_hbm.at[0, pl.ds(0, lens[p])],
                              ssems[p]).wait()


def _make_kernel(batch):
    mesh = plsc.VectorSubcoreMesh(core_axis_name="c", subcore_axis_name="s")
    rows = batch // NW
    return pl.kernel(
        _body,
        out_type=jax.ShapeDtypeStruct((batch, SEQ_LEN, EMBED_DIM), jnp.float32),
        mesh=mesh,
        scratch_types=[
            pltpu.VMEM((rows, SEQ_LEN), jnp.int32),
            pltpu.VMEM((SEQ_LEN, EMBED_DIM), jnp.float32),
            pltpu.VMEM((112,), jnp.int32),
            pltpu.VMEM((H1,), jnp.int32),
            pltpu.VMEM((H0, 2 * EMBED_DIM), jnp.float32),
            pltpu.VMEM((H1, 2 * EMBED_DIM), jnp.float32),
            pltpu.VMEM((H0, EMBED_DIM), jnp.float32),
            pltpu.VMEM((H1, EMBED_DIM), jnp.float32),
            pltpu.SemaphoreType.DMA,
            pltpu.SemaphoreType.DMA,
            pltpu.SemaphoreType.DMA,
            pltpu.SemaphoreType.DMA,
        ],
        compiler_params=pltpu.CompilerParams(use_tc_tiling_on_sc=True),
    )


def kernel(x, token_table, pos_table):
    batch, seq = x.shape
    assert seq == SEQ_LEN
    vocab, dim = token_table.shape
    assert dim == EMBED_DIM
    tok_pairs = token_table.reshape(vocab // 2, 2 * EMBED_DIM)
    run = _make_kernel(batch)
    return run(x.astype(jnp.int32), pos_table, tok_pairs)
